# Initial kernel scaffold; baseline (speedup 1.0000x reference)
import jax
import jax.numpy as jnp
from jax.experimental import pallas as pl


def kernel(inputs, table):
    raise NotImplementedError("write your pallas kernel here")


# speedup vs baseline: 3.0070x; 3.0070x over previous
import jax
import jax.numpy as jnp
from jax import lax
from jax.experimental import pallas as pl
from jax.experimental.pallas import tpu as pltpu
from jax.experimental.pallas import tpu_sc as plsc

NC = 2
NS = 16
NW = NC * NS
CHUNK = 128


def _make_body(B, L, D, BPW, NCHUNK, NSTREAM):
    def body(idx_hbm, table_hbm, out_hbm, idx_v, acc_v, sem):
        wid = lax.axis_index("s") * NC + lax.axis_index("c")
        pltpu.sync_copy(idx_hbm.at[wid], idx_v)

        zeros = jnp.zeros((16,), jnp.float32)

        def zero_row(b, carry):
            for h in range(D // 16):
                acc_v[b, pl.ds(h * 16, 16)] = zeros
            return carry

        lax.fori_loop(0, BPW, zero_row, 0)

        def fire(r, carry):
            c = lax.rem(r, NCHUNK)
            pltpu.async_copy(
                table_hbm.at[idx_v.at[r]],
                acc_v.at[pl.ds(c * CHUNK, CHUNK)],
                sem,
                add=True,
            )
            return carry

        lax.fori_loop(0, NSTREAM, fire, 0)

        def drain(r, carry):
            pltpu.make_async_copy(
                table_hbm.at[idx_v.at[0]],
                acc_v.at[pl.ds(0, CHUNK)],
                sem,
            ).wait()
            return carry

        lax.fori_loop(0, NSTREAM, drain, 0)

        scale = jnp.float32(1.0 / L)

        def scale_row(b, carry):
            for h in range(D // 16):
                acc_v[b, pl.ds(h * 16, 16)] = acc_v[b, pl.ds(h * 16, 16)] * scale
            return carry

        lax.fori_loop(0, BPW, scale_row, 0)
        pltpu.sync_copy(acc_v, out_hbm.at[pl.ds(wid * BPW, BPW)])

    return body


def kernel(inputs, table):
    B, L = inputs.shape
    V, D = table.shape
    BPW = B // NW
    NCHUNK = BPW // CHUNK
    NSTREAM = L * NCHUNK

    idx = (
        inputs.astype(jnp.int32)
        .reshape(NW, NCHUNK, CHUNK, L)
        .transpose(0, 3, 1, 2)
        .reshape(NW, NSTREAM, CHUNK)
    )

    mesh = plsc.VectorSubcoreMesh(
        core_axis_name="c", subcore_axis_name="s", num_cores=NC, num_subcores=NS
    )
    f = pl.kernel(
        _make_body(B, L, D, BPW, NCHUNK, NSTREAM),
        out_type=jax.ShapeDtypeStruct((B, D), jnp.float32),
        mesh=mesh,
        scratch_types=[
            pltpu.VMEM((NSTREAM, CHUNK), jnp.int32),
            pltpu.VMEM((BPW, D), jnp.float32),
            pltpu.SemaphoreType.DMA,
        ],
        compiler_params=pltpu.CompilerParams(use_tc_tiling_on_sc=False),
    )
    return f(idx, table)
